# baseline (speedup 1.0000x reference)
import functools
import jax
import jax.numpy as jnp
from jax.experimental import pallas as pl
from jax.experimental.pallas import tpu as pltpu
from jax.experimental.pallas import tpu_sc as plsc

F32 = jnp.float32
BF16 = jnp.bfloat16
I32 = jnp.int32

K = 4
BLK = 256
RW = 128


def _router_body(x_ref, rw_ref, ws_ref, ds_ref, meta_ref, *, E, T, NB):
    x = x_ref[...]
    rw = rw_ref[...]
    logits = jax.lax.dot_general(x, rw, (((1,), (1,)), ((), ())),
                                 preferred_element_type=F32)
    m = jnp.max(logits, axis=1, keepdims=True)
    ex = jnp.exp(logits - m)
    probs = ex / jnp.sum(ex, axis=1, keepdims=True)

    iota = jax.lax.broadcasted_iota(I32, (T, E), 1)
    masked = probs
    selected = jnp.zeros((T, E), dtype=jnp.bool_)
    firsts = []
    for _ in range(K):
        mx = jnp.max(masked, axis=1, keepdims=True)
        is_max = masked == mx
        cand = jnp.where(is_max, iota, E)
        first = jnp.min(cand, axis=1, keepdims=True)
        firsts.append(first)
        newly = iota == first
        selected = jnp.logical_or(selected, newly)
        masked = jnp.where(newly, -jnp.inf, masked)

    self32 = selected.astype(F32)
    psel = jnp.where(selected, probs, 0.0)
    wsum = jnp.sum(psel, axis=1, keepdims=True)
    wnorm = psel / wsum

    ranks_parts = []
    CH = 256
    for i in range(T // CH):
        row = jax.lax.broadcasted_iota(I32, (CH, T), 0) + i * CH
        col = jax.lax.broadcasted_iota(I32, (CH, T), 1)
        ltri = (col < row).astype(F32)
        ranks_parts.append(jax.lax.dot_general(
            ltri, self32, (((1,), (0,)), ((), ())),
            preferred_element_type=F32))
    ranks = jnp.concatenate(ranks_parts, axis=0)

    counts = jnp.sum(self32, axis=0, keepdims=True)
    cnt = counts.astype(I32)
    pc = ((cnt + (BLK - 1)) // BLK) * BLK
    r16 = jax.lax.broadcasted_iota(I32, (E, E), 0)
    c16 = jax.lax.broadcasted_iota(I32, (E, E), 1)
    l16 = (r16 < c16).astype(F32)
    offs = jax.lax.dot_general(pc.astype(F32), l16, (((1,), (0,)), ((), ())),
                               preferred_element_type=F32).astype(I32)

    dest = offs + ranks.astype(I32)

    col8 = jax.lax.broadcasted_iota(I32, (T, 2 * K), 1)
    ws = jnp.zeros((T, 2 * K), F32)
    ds = jnp.zeros((T, 2 * K), I32)
    for r in range(K):
        first = firsts[r]
        w_r = jnp.sum(jnp.where(iota == first, wnorm, 0.0), axis=1,
                      keepdims=True)
        d_r = jnp.sum(jnp.where(iota == first, dest, 0), axis=1,
                      keepdims=True)
        ws = jnp.where(col8 == r, w_r, ws)
        ds = jnp.where(col8 == r, d_r, ds)
    ws_ref[...] = ws
    ds_ref[...] = ds

    tp = offs[0, E - 1] + pc[0, E - 1]
    bcol = jax.lax.broadcasted_iota(I32, (8, 64), 1)
    row_start = bcol * BLK
    def expert_of(rows):
        acc = jnp.zeros((8, 64), I32)
        for e in range(E):
            inr = jnp.logical_and(rows >= offs[0, e], rows < offs[0, e] + pc[0, e])
            acc = acc + jnp.where(inr, e, 0)
        return acc

    be_raw = expert_of(row_start)
    be_last = expert_of(jnp.full((8, 64), tp - BLK, I32))
    active = row_start < tp
    be_fill = jnp.where(active, be_raw, be_last)

    prev = jnp.concatenate([be_fill[:, :1], be_fill[:, :-1]], axis=1)
    ff = jnp.where(bcol == 0, 1, (be_fill != prev).astype(I32))
    ri64 = jax.lax.broadcasted_iota(I32, (64, 64), 0)
    ci64 = jax.lax.broadcasted_iota(I32, (64, 64), 1)
    m64 = (ri64 <= ci64).astype(F32)
    ordc = jax.lax.dot_general(ff.astype(F32), m64, (((1,), (0,)), ((), ())),
                               preferred_element_type=F32).astype(I32)
    slot = (ordc - 1) % 2
    pcpos = (pc[0] > 0)
    nxt = []
    for e in range(E):
        cand = jnp.where(jnp.logical_and(c16[0] > e, pcpos), c16[0], 99)
        mn = jnp.min(cand)
        nxt.append(jnp.where(mn == 99, -1, mn))
    ne = jnp.full((8, 64), -1, I32)
    for e in range(E):
        ne = jnp.where(be_fill == e, nxt[e], ne)

    acti = active.astype(I32)
    nxtff = jnp.concatenate([ff[:, 1:], jnp.ones((8, 1), I32)], axis=1)
    nact = jnp.concatenate([acti[:, 1:], jnp.zeros((8, 1), I32)], axis=1)
    lf = jnp.where(jnp.logical_and(active, jnp.logical_or(nxtff == 1,
                                                          nact == 0)), 1, 0)
    nxt2 = []
    for e in range(E):
        n1 = nxt[e]
        v = jnp.asarray(-1, I32)
        for e2 in range(E):
            v = jnp.where(n1 == e2, nxt[e2], v)
        nxt2.append(v)
    ne2 = jnp.full((8, 64), -1, I32)
    for e in range(E):
        ne2 = jnp.where(be_fill == e, nxt2[e], ne2)

    rowi = jax.lax.broadcasted_iota(I32, (8, 64), 0)
    meta = jnp.where(rowi == 0, be_fill, 0)
    meta = jnp.where(rowi == 1, active.astype(I32), meta)
    meta = jnp.where(rowi == 2, ff, meta)
    meta = jnp.where(rowi == 3, slot, meta)
    meta = jnp.where(rowi == 4, ne, meta)
    meta = jnp.where(rowi == 5, lf, meta)
    meta = jnp.where(rowi == 6, ne2, meta)
    meta_ref[...] = meta


def _make_router(E, T, NB):
    return pl.pallas_call(
        functools.partial(_router_body, E=E, T=T, NB=NB),
        out_shape=[
            jax.ShapeDtypeStruct((T, 2 * K), F32),
            jax.ShapeDtypeStruct((T, 2 * K), I32),
            jax.ShapeDtypeStruct((8, 64), I32),
        ],
    )


NW = 32


def _scatter_rows(x, dest_flat, P, H):
    T = x.shape[0]
    mesh = plsc.VectorSubcoreMesh(core_axis_name="core",
                                  subcore_axis_name="subcore")
    nch = K * T // RW
    per_w = nch // NW
    ntch = T // RW

    @functools.partial(
        pl.kernel,
        out_type=jax.ShapeDtypeStruct((P, H), F32),
        mesh=mesh,
        scratch_types=[
            pltpu.VMEM((1, RW), I32),
            pltpu.VMEM((RW, H), F32),
            pltpu.SemaphoreType.DMA,
        ],
    )
    def scat(x_hbm, i_hbm, o_hbm, idx_v, buf_v, sem):
        cid = jax.lax.axis_index("core")
        sid = jax.lax.axis_index("subcore")
        wid = sid * 2 + cid

        @pl.loop(0, per_w)
        def _(j):
            c = wid * per_w + j
            tch = c % ntch
            pltpu.sync_copy(i_hbm.at[:, pl.ds(c * RW, RW)], idx_v)
            pltpu.sync_copy(x_hbm.at[pl.ds(tch * RW, RW), :], buf_v)
            pltpu.async_copy(buf_v, o_hbm.at[idx_v.at[0]], sem).wait()

    return scat(x, dest_flat)


def _gather_rows(y, dest_flat, H):
    A = dest_flat.shape[1]
    mesh = plsc.VectorSubcoreMesh(core_axis_name="core",
                                  subcore_axis_name="subcore")
    nch = A // RW
    per_w = nch // NW

    @functools.partial(
        pl.kernel,
        out_type=jax.ShapeDtypeStruct((A, H), F32),
        mesh=mesh,
        scratch_types=[
            pltpu.VMEM((1, RW), I32),
            pltpu.VMEM((RW, H), F32),
            pltpu.SemaphoreType.DMA,
        ],
    )
    def gath(y_hbm, i_hbm, o_hbm, idx_v, buf_v, sem):
        cid = jax.lax.axis_index("core")
        sid = jax.lax.axis_index("subcore")
        wid = sid * 2 + cid

        @pl.loop(0, per_w)
        def _(j):
            c = wid * per_w + j
            pltpu.sync_copy(i_hbm.at[:, pl.ds(c * RW, RW)], idx_v)
            pltpu.async_copy(y_hbm.at[idx_v.at[0]], buf_v, sem).wait()
            pltpu.sync_copy(buf_v, o_hbm.at[pl.ds(c * RW, RW), :])

    return gath(y, dest_flat)


def _weight_pipeline(meta_refs, b, w_ref, wraw_ref, wbf_ref, sem_ref):
    (be_ref, act_ref, ff_ref, slot_ref, ne_ref, lf_ref, ne2_ref) = meta_refs

    @pl.when(b == 0)
    def _():
        pltpu.make_async_copy(w_ref.at[be_ref[0]], wraw_ref.at[0],
                              sem_ref.at[0]).start()
        pltpu.make_async_copy(w_ref.at[be_ref[0]], wraw_ref.at[0],
                              sem_ref.at[0]).wait()
        wbf_ref[0] = wraw_ref[0].astype(BF16)

        @pl.when(ne_ref[0] >= 0)
        def _():
            pltpu.make_async_copy(w_ref.at[ne_ref[0]], wraw_ref.at[1],
                                  sem_ref.at[1]).start()

    @pl.when(jnp.logical_and(lf_ref[b] == 1, ne_ref[b] >= 0))
    def _():
        def stage_next(s):
            pltpu.make_async_copy(w_ref.at[ne_ref[b]], wraw_ref.at[1 - s],
                                  sem_ref.at[1 - s]).wait()

            @pl.when(ne2_ref[b] >= 0)
            def _():
                pltpu.make_async_copy(w_ref.at[ne2_ref[b]], wraw_ref.at[s],
                                      sem_ref.at[s]).start()

            wbf_ref[1 - s] = wraw_ref[1 - s].astype(BF16)

        @pl.when(slot_ref[b] == 0)
        def _():
            stage_next(0)

        @pl.when(slot_ref[b] == 1)
        def _():
            stage_next(1)


def _compute_slotted(slot_ref, b, act_ref, wbf_ref, fn):
    @pl.when(jnp.logical_and(act_ref[b] == 1, slot_ref[b] == 0))
    def _():
        fn(wbf_ref[0])

    @pl.when(jnp.logical_and(act_ref[b] == 1, slot_ref[b] == 1))
    def _():
        fn(wbf_ref[1])


def _l1_body(be_ref, act_ref, ff_ref, slot_ref, ne_ref, lf_ref, ne2_ref,
             x_ref, w_ref, o_ref, wraw_ref, wbf_ref, sem_ref):
    b = pl.program_id(0)
    meta = (be_ref, act_ref, ff_ref, slot_ref, ne_ref, lf_ref, ne2_ref)
    _weight_pipeline(meta, b, w_ref, wraw_ref, wbf_ref, sem_ref)

    def fn(wb):
        h = jax.lax.dot_general(x_ref[...].astype(BF16), wb,
                                (((1,), (1,)), ((), ())),
                                preferred_element_type=F32)
        o_ref[...] = (h * jax.nn.sigmoid(h)).astype(BF16)

    _compute_slotted(slot_ref, b, act_ref, wbf_ref, fn)


def _l2_body(be_ref, act_ref, ff_ref, slot_ref, ne_ref, lf_ref, ne2_ref,
             h_ref, w_ref, o_ref, wraw_ref, wbf_ref, sem_ref):
    b = pl.program_id(0)
    meta = (be_ref, act_ref, ff_ref, slot_ref, ne_ref, lf_ref, ne2_ref)
    _weight_pipeline(meta, b, w_ref, wraw_ref, wbf_ref, sem_ref)

    def fn(wb):
        h = jax.lax.dot_general(h_ref[...], wb, (((1,), (1,)), ((), ())),
                                preferred_element_type=F32)
        o_ref[...] = (h * jax.nn.sigmoid(h)).astype(BF16)

    _compute_slotted(slot_ref, b, act_ref, wbf_ref, fn)


def _l3_body(be_ref, act_ref, ff_ref, slot_ref, ne_ref, lf_ref, ne2_ref,
             h_ref, w_ref, o_ref, wraw_ref, wbf_ref, sem_ref):
    b = pl.program_id(0)
    meta = (be_ref, act_ref, ff_ref, slot_ref, ne_ref, lf_ref, ne2_ref)
    _weight_pipeline(meta, b, w_ref, wraw_ref, wbf_ref, sem_ref)

    def fn(wb):
        y = jax.lax.dot_general(h_ref[...], wb, (((1,), (1,)), ((), ())),
                                preferred_element_type=F32)
        o_ref[...] = y

    _compute_slotted(slot_ref, b, act_ref, wbf_ref, fn)


def _layer_pass(body, xin, w, meta_rows, NB, out_cols, out_dtype):
    P = NB * BLK
    Wd, Wk = w.shape[1], w.shape[2]
    return pl.pallas_call(
        body,
        grid_spec=pltpu.PrefetchScalarGridSpec(
            num_scalar_prefetch=7,
            grid=(NB,),
            in_specs=[
                pl.BlockSpec((BLK, xin.shape[1]),
                             lambda b, *refs: (b, 0)),
                pl.BlockSpec(memory_space=pl.ANY),
            ],
            out_specs=pl.BlockSpec((BLK, out_cols), lambda b, *refs: (b, 0)),
            scratch_shapes=[
                pltpu.VMEM((2, Wd, Wk), F32),
                pltpu.VMEM((2, Wd, Wk), BF16),
                pltpu.SemaphoreType.DMA((2,)),
            ],
        ),
        out_shape=jax.ShapeDtypeStruct((P, out_cols), out_dtype),
    )(*meta_rows, xin, w)


def _combine_body(y_ref, w_ref, o_ref):
    y4 = y_ref[...]
    wn = w_ref[...]
    acc = wn[:, 0:1] * y4[0]
    for k in range(1, K):
        acc = acc + wn[:, k:k + 1] * y4[k]
    o_ref[...] = acc


def kernel(hidden_states, router_weight, w1, w2, w3):
    B, S, H = hidden_states.shape
    E, F, _ = w1.shape
    T = B * S
    P = T * K + E * BLK
    NB = P // BLK
    TB = 256

    x = hidden_states.reshape(T, H)

    w_slot, dest_slot, meta = _make_router(E, T, NB)(x, router_weight)

    meta_rows = [meta[r, :NB] for r in range(7)]
    dest_flat = dest_slot[:, :K].T.reshape(1, K * T)

    x_sorted = _scatter_rows(x, dest_flat, P, H)

    h1 = _layer_pass(_l1_body, x_sorted, w1, meta_rows, NB, F, BF16)
    h2 = _layer_pass(_l2_body, h1, w2, meta_rows, NB, F, BF16)
    y_sorted = _layer_pass(_l3_body, h2, w3, meta_rows, NB, H, F32)

    ygat = _gather_rows(y_sorted, dest_flat, H).reshape(K, T, H)

    out = pl.pallas_call(
        _combine_body,
        grid=(T // TB,),
        in_specs=[
            pl.BlockSpec((K, TB, H), lambda t: (0, t, 0)),
            pl.BlockSpec((TB, 2 * K), lambda t: (t, 0)),
        ],
        out_specs=pl.BlockSpec((TB, H), lambda t: (t, 0)),
        out_shape=jax.ShapeDtypeStruct((T, H), F32),
    )(ygat, w_slot)

    return out.reshape(B, S, H)


# speedup vs baseline: 1.0026x; 1.0026x over previous
import functools
import jax
import jax.numpy as jnp
from jax.experimental import pallas as pl
from jax.experimental.pallas import tpu as pltpu
from jax.experimental.pallas import tpu_sc as plsc

F32 = jnp.float32
BF16 = jnp.bfloat16
I32 = jnp.int32

K = 4
BLK = 256
RW = 128


def _router_body(x_ref, rw_ref, ws_ref, ds_ref, meta_ref, *, E, T, NB):
    x = x_ref[...]
    rw = rw_ref[...]
    logits = jax.lax.dot_general(x, rw, (((1,), (1,)), ((), ())),
                                 preferred_element_type=F32)
    m = jnp.max(logits, axis=1, keepdims=True)
    ex = jnp.exp(logits - m)
    probs = ex / jnp.sum(ex, axis=1, keepdims=True)

    iota = jax.lax.broadcasted_iota(I32, (T, E), 1)
    masked = probs
    selected = jnp.zeros((T, E), dtype=jnp.bool_)
    firsts = []
    for _ in range(K):
        mx = jnp.max(masked, axis=1, keepdims=True)
        is_max = masked == mx
        cand = jnp.where(is_max, iota, E)
        first = jnp.min(cand, axis=1, keepdims=True)
        firsts.append(first)
        newly = iota == first
        selected = jnp.logical_or(selected, newly)
        masked = jnp.where(newly, -jnp.inf, masked)

    self32 = selected.astype(F32)
    psel = jnp.where(selected, probs, 0.0)
    wsum = jnp.sum(psel, axis=1, keepdims=True)
    wnorm = psel / wsum

    ranks_parts = []
    CH = 256
    for i in range(T // CH):
        row = jax.lax.broadcasted_iota(I32, (CH, T), 0) + i * CH
        col = jax.lax.broadcasted_iota(I32, (CH, T), 1)
        ltri = (col < row).astype(F32)
        ranks_parts.append(jax.lax.dot_general(
            ltri, self32, (((1,), (0,)), ((), ())),
            preferred_element_type=F32))
    ranks = jnp.concatenate(ranks_parts, axis=0)

    counts = jnp.sum(self32, axis=0, keepdims=True)
    cnt = counts.astype(I32)
    pc = ((cnt + (BLK - 1)) // BLK) * BLK
    r16 = jax.lax.broadcasted_iota(I32, (E, E), 0)
    c16 = jax.lax.broadcasted_iota(I32, (E, E), 1)
    l16 = (r16 < c16).astype(F32)
    offs = jax.lax.dot_general(pc.astype(F32), l16, (((1,), (0,)), ((), ())),
                               preferred_element_type=F32).astype(I32)

    dest = offs + ranks.astype(I32)

    col8 = jax.lax.broadcasted_iota(I32, (T, 2 * K), 1)
    ws = jnp.zeros((T, 2 * K), F32)
    ds = jnp.zeros((T, 2 * K), I32)
    for r in range(K):
        first = firsts[r]
        w_r = jnp.sum(jnp.where(iota == first, wnorm, 0.0), axis=1,
                      keepdims=True)
        d_r = jnp.sum(jnp.where(iota == first, dest, 0), axis=1,
                      keepdims=True)
        ws = jnp.where(col8 == r, w_r, ws)
        ds = jnp.where(col8 == r, d_r, ds)
    ws_ref[...] = ws
    ds_ref[...] = ds

    tp = offs[0, E - 1] + pc[0, E - 1]
    bcol = jax.lax.broadcasted_iota(I32, (8, 64), 1)
    row_start = bcol * BLK
    def expert_of(rows):
        acc = jnp.zeros((8, 64), I32)
        for e in range(E):
            inr = jnp.logical_and(rows >= offs[0, e], rows < offs[0, e] + pc[0, e])
            acc = acc + jnp.where(inr, e, 0)
        return acc

    be_raw = expert_of(row_start)
    be_last = expert_of(jnp.full((8, 64), tp - BLK, I32))
    active = row_start < tp
    be_fill = jnp.where(active, be_raw, be_last)

    prev = jnp.concatenate([be_fill[:, :1], be_fill[:, :-1]], axis=1)
    ff = jnp.where(bcol == 0, 1, (be_fill != prev).astype(I32))
    ri64 = jax.lax.broadcasted_iota(I32, (64, 64), 0)
    ci64 = jax.lax.broadcasted_iota(I32, (64, 64), 1)
    m64 = (ri64 <= ci64).astype(F32)
    ordc = jax.lax.dot_general(ff.astype(F32), m64, (((1,), (0,)), ((), ())),
                               preferred_element_type=F32).astype(I32)
    slot = (ordc - 1) % 2
    pcpos = (pc[0] > 0)
    nxt = []
    for e in range(E):
        cand = jnp.where(jnp.logical_and(c16[0] > e, pcpos), c16[0], 99)
        mn = jnp.min(cand)
        nxt.append(jnp.where(mn == 99, -1, mn))
    ne = jnp.full((8, 64), -1, I32)
    for e in range(E):
        ne = jnp.where(be_fill == e, nxt[e], ne)

    rowi = jax.lax.broadcasted_iota(I32, (8, 64), 0)
    meta = jnp.where(rowi == 0, be_fill, 0)
    meta = jnp.where(rowi == 1, active.astype(I32), meta)
    meta = jnp.where(rowi == 2, ff, meta)
    meta = jnp.where(rowi == 3, slot, meta)
    meta = jnp.where(rowi == 4, ne, meta)
    meta_ref[...] = meta


def _make_router(E, T, NB):
    return pl.pallas_call(
        functools.partial(_router_body, E=E, T=T, NB=NB),
        out_shape=[
            jax.ShapeDtypeStruct((T, 2 * K), F32),
            jax.ShapeDtypeStruct((T, 2 * K), I32),
            jax.ShapeDtypeStruct((8, 64), I32),
        ],
    )


NW = 32


def _scatter_rows(x, dest_flat, P, H):
    T = x.shape[0]
    mesh = plsc.VectorSubcoreMesh(core_axis_name="core",
                                  subcore_axis_name="subcore")
    nch = K * T // RW
    per_w = nch // NW
    ntch = T // RW

    @functools.partial(
        pl.kernel,
        out_type=jax.ShapeDtypeStruct((P, H), F32),
        mesh=mesh,
        scratch_types=[
            pltpu.VMEM((1, RW), I32),
            pltpu.VMEM((RW, H), F32),
            pltpu.SemaphoreType.DMA,
        ],
    )
    def scat(x_hbm, i_hbm, o_hbm, idx_v, buf_v, sem):
        cid = jax.lax.axis_index("core")
        sid = jax.lax.axis_index("subcore")
        wid = sid * 2 + cid

        @pl.loop(0, per_w)
        def _(j):
            c = wid * per_w + j
            tch = c % ntch
            pltpu.sync_copy(i_hbm.at[:, pl.ds(c * RW, RW)], idx_v)
            pltpu.sync_copy(x_hbm.at[pl.ds(tch * RW, RW), :], buf_v)
            pltpu.async_copy(buf_v, o_hbm.at[idx_v.at[0]], sem).wait()

    return scat(x, dest_flat)


def _gather_rows(y, dest_flat, H):
    A = dest_flat.shape[1]
    mesh = plsc.VectorSubcoreMesh(core_axis_name="core",
                                  subcore_axis_name="subcore")
    nch = A // RW
    per_w = nch // NW

    @functools.partial(
        pl.kernel,
        out_type=jax.ShapeDtypeStruct((A, H), F32),
        mesh=mesh,
        scratch_types=[
            pltpu.VMEM((1, RW), I32),
            pltpu.VMEM((RW, H), F32),
            pltpu.SemaphoreType.DMA,
        ],
    )
    def gath(y_hbm, i_hbm, o_hbm, idx_v, buf_v, sem):
        cid = jax.lax.axis_index("core")
        sid = jax.lax.axis_index("subcore")
        wid = sid * 2 + cid

        @pl.loop(0, per_w)
        def _(j):
            c = wid * per_w + j
            pltpu.sync_copy(i_hbm.at[:, pl.ds(c * RW, RW)], idx_v)
            pltpu.async_copy(y_hbm.at[idx_v.at[0]], buf_v, sem).wait()
            pltpu.sync_copy(buf_v, o_hbm.at[pl.ds(c * RW, RW), :])

    return gath(y, dest_flat)


def _weight_pipeline(meta_refs, b, w_ref, wraw_ref, wbf_ref, sem_ref):
    be_ref, act_ref, ff_ref, slot_ref, ne_ref = meta_refs

    @pl.when(b == 0)
    def _():
        pltpu.make_async_copy(w_ref.at[be_ref[0]], wraw_ref.at[0],
                              sem_ref.at[0]).start()

    @pl.when(jnp.logical_and(act_ref[b] == 1, ff_ref[b] == 1))
    def _():
        def arrive_and_prefetch(s):
            pltpu.make_async_copy(w_ref.at[be_ref[b]], wraw_ref.at[s],
                                  sem_ref.at[s]).wait()

            @pl.when(ne_ref[b] >= 0)
            def _():
                pltpu.make_async_copy(w_ref.at[ne_ref[b]],
                                      wraw_ref.at[1 - s],
                                      sem_ref.at[1 - s]).start()

            wbf_ref[...] = wraw_ref[s].astype(BF16)

        @pl.when(slot_ref[b] == 0)
        def _():
            arrive_and_prefetch(0)

        @pl.when(slot_ref[b] == 1)
        def _():
            arrive_and_prefetch(1)


def _l1_body(be_ref, act_ref, ff_ref, slot_ref, ne_ref,
             x_ref, w_ref, o_ref, wraw_ref, wbf_ref, sem_ref):
    b = pl.program_id(0)
    _weight_pipeline((be_ref, act_ref, ff_ref, slot_ref, ne_ref),
                     b, w_ref, wraw_ref, wbf_ref, sem_ref)

    @pl.when(act_ref[b] == 1)
    def _():
        h = jax.lax.dot_general(x_ref[...].astype(BF16), wbf_ref[...],
                                (((1,), (1,)), ((), ())),
                                preferred_element_type=F32)
        o_ref[...] = (h * jax.nn.sigmoid(h)).astype(BF16)


def _l2_body(be_ref, act_ref, ff_ref, slot_ref, ne_ref,
             h_ref, w_ref, o_ref, wraw_ref, wbf_ref, sem_ref):
    b = pl.program_id(0)
    _weight_pipeline((be_ref, act_ref, ff_ref, slot_ref, ne_ref),
                     b, w_ref, wraw_ref, wbf_ref, sem_ref)

    @pl.when(act_ref[b] == 1)
    def _():
        h = jax.lax.dot_general(h_ref[...], wbf_ref[...],
                                (((1,), (1,)), ((), ())),
                                preferred_element_type=F32)
        o_ref[...] = (h * jax.nn.sigmoid(h)).astype(BF16)


def _l3_body(be_ref, act_ref, ff_ref, slot_ref, ne_ref,
             h_ref, w_ref, o_ref, wraw_ref, wbf_ref, sem_ref):
    b = pl.program_id(0)
    _weight_pipeline((be_ref, act_ref, ff_ref, slot_ref, ne_ref),
                     b, w_ref, wraw_ref, wbf_ref, sem_ref)

    @pl.when(act_ref[b] == 1)
    def _():
        y = jax.lax.dot_general(h_ref[...], wbf_ref[...],
                                (((1,), (1,)), ((), ())),
                                preferred_element_type=F32)
        o_ref[...] = y


def _layer_pass(body, xin, w, meta_rows, NB, out_cols, out_dtype):
    P = NB * BLK
    Wd, Wk = w.shape[1], w.shape[2]
    return pl.pallas_call(
        body,
        grid_spec=pltpu.PrefetchScalarGridSpec(
            num_scalar_prefetch=5,
            grid=(NB,),
            in_specs=[
                pl.BlockSpec((BLK, xin.shape[1]),
                             lambda b, *refs: (b, 0)),
                pl.BlockSpec(memory_space=pl.ANY),
            ],
            out_specs=pl.BlockSpec((BLK, out_cols), lambda b, *refs: (b, 0)),
            scratch_shapes=[
                pltpu.VMEM((2, Wd, Wk), F32),
                pltpu.VMEM((Wd, Wk), BF16),
                pltpu.SemaphoreType.DMA((2,)),
            ],
        ),
        out_shape=jax.ShapeDtypeStruct((P, out_cols), out_dtype),
    )(*meta_rows, xin, w)


def _combine_body(y_ref, w_ref, o_ref):
    y4 = y_ref[...]
    wn = w_ref[...]
    acc = wn[:, 0:1] * y4[0]
    for k in range(1, K):
        acc = acc + wn[:, k:k + 1] * y4[k]
    o_ref[...] = acc


def kernel(hidden_states, router_weight, w1, w2, w3):
    B, S, H = hidden_states.shape
    E, F, _ = w1.shape
    T = B * S
    P = T * K + E * BLK
    NB = P // BLK
    TB = 256

    x = hidden_states.reshape(T, H)

    w_slot, dest_slot, meta = _make_router(E, T, NB)(x, router_weight)

    meta_rows = [meta[r, :NB] for r in range(5)]
    dest_flat = dest_slot[:, :K].T.reshape(1, K * T)

    x_sorted = _scatter_rows(x, dest_flat, P, H)

    h1 = _layer_pass(_l1_body, x_sorted, w1, meta_rows, NB, F, BF16)
    h2 = _layer_pass(_l2_body, h1, w2, meta_rows, NB, F, BF16)
    y_sorted = _layer_pass(_l3_body, h2, w3, meta_rows, NB, H, F32)

    ygat = _gather_rows(y_sorted, dest_flat, H).reshape(K, T, H)

    out = pl.pallas_call(
        _combine_body,
        grid=(T // TB,),
        in_specs=[
            pl.BlockSpec((K, TB, H), lambda t: (0, t, 0)),
            pl.BlockSpec((TB, 2 * K), lambda t: (t, 0)),
        ],
        out_specs=pl.BlockSpec((TB, H), lambda t: (t, 0)),
        out_shape=jax.ShapeDtypeStruct((T, H), F32),
    )(ygat, w_slot)

    return out.reshape(B, S, H)
